# baseline (speedup 1.0000x reference)
import functools

import jax
import jax.numpy as jnp
from jax import lax
from jax.experimental import pallas as pl
from jax.experimental.pallas import tpu as pltpu
from jax.experimental.pallas import tpu_sc as plsc

N_NODES = 10000
N_EDGES = 320000
D = 64
NC = 2
NS = 16
NW = NC * NS
EPW = N_EDGES // NW
CH = 80
NCHUNK = EPW // CH
RPT = N_NODES // NS
NBUF = 4
UNROLL = 8

_mesh = plsc.VectorSubcoreMesh(
    core_axis_name="c", subcore_axis_name="s", num_cores=NC, num_subcores=NS)


def _seg_body(y_hbm, src_hbm, dst_hbm, w_hbm, out_hbm,
              src_v, w_v, dsti_v, *rest):
    gbuf = list(rest[:NBUF])
    fbuf = list(rest[NBUF:2 * NBUF])
    acc = rest[2 * NBUF]
    gsem = list(rest[2 * NBUF + 1:3 * NBUF + 1])
    ssem = list(rest[3 * NBUF + 1:4 * NBUF + 1])
    del rest
    c = lax.axis_index("c")
    s = lax.axis_index("s")
    wid = c * NS + s
    base = wid * EPW

    zvec = jnp.zeros((16,), jnp.float32)
    zbuf = fbuf[0]

    @pl.loop(0, CH)
    def _(r):
        for dd in range(D // 16):
            zbuf[r, pl.ds(dd * 16, 16)] = zvec

    @pl.loop(0, RPT // CH)
    def _(k):
        pltpu.sync_copy(zbuf, acc.at[pl.ds(s * RPT + k * CH, CH)])

    pltpu.sync_copy(zbuf.at[pl.ds(0, RPT % CH)],
                    acc.at[pl.ds(s * RPT + (RPT // CH) * CH, RPT % CH)])

    pltpu.sync_copy(src_hbm.at[pl.ds(base, EPW)], src_v)
    pltpu.sync_copy(w_hbm.at[pl.ds(base, EPW)], w_v)
    pltpu.sync_copy(dst_hbm.at[wid], dsti_v)

    plsc.subcore_barrier()

    def start_gather(k, b):
        pltpu.async_copy(y_hbm.at[src_v.at[pl.ds(k * CH, CH)]], gbuf[b],
                         gsem[b])

    def wait_gather(k, b):
        pltpu.make_async_copy(y_hbm.at[src_v.at[pl.ds(k * CH, CH)]], gbuf[b],
                              gsem[b]).wait()

    def start_scatter(k, b):
        pltpu.async_copy(fbuf[b], acc.at[dsti_v.at[k]], ssem[b], add=True)

    def wait_scatter(k, b):
        pltpu.make_async_copy(fbuf[b], acc.at[dsti_v.at[k]], ssem[b]).wait()

    def multiply(k, b):
        gb = gbuf[b]
        fb = fbuf[b]

        @plsc.parallel_loop(0, CH, unroll=UNROLL)
        def _(r):
            wb = plsc.load_gather(
                w_v, [jnp.full((16,), k * CH + r, dtype=jnp.int32)])
            for g2 in range(D // 32):
                pair = gb[r, pl.ds(g2 * 32, 32)]
                p0, p1 = plsc.unpack(pair, format=plsc.PackFormat.INTERLEAVED,
                                     preferred_element_type=jnp.float32)
                fb[r, pl.ds(g2 * 32, 16)] = p0 * wb
                fb[r, pl.ds(g2 * 32 + 16, 16)] = p1 * wb

    for b in range(NBUF - 1):
        start_gather(b, b)

    ngrp = (NCHUNK + NBUF - 1) // NBUF

    @pl.loop(0, ngrp)
    def _(j):
        for i in range(NBUF):
            k = j * NBUF + i

            @pl.when(k < NCHUNK)
            def _():
                wait_gather(k, i)
                multiply(k, i)
                start_scatter(k, i)

            @pl.when(jnp.logical_and(k >= 1, k <= NCHUNK - 1))
            def _():
                wait_scatter(k - 1, (i - 1) % NBUF)

            @pl.when(k + NBUF - 1 < NCHUNK)
            def _():
                start_gather(k + NBUF - 1, (i + NBUF - 1) % NBUF)

    wait_scatter(NCHUNK - 1, (NCHUNK - 1) % NBUF)

    plsc.subcore_barrier()
    pltpu.sync_copy(acc.at[pl.ds(s * RPT, RPT)],
                    out_hbm.at[c, pl.ds(s * RPT, RPT)])


_seg_sum = pl.kernel(
    _seg_body,
    out_type=jax.ShapeDtypeStruct((NC, N_NODES, D), jnp.float32),
    mesh=_mesh,
    scratch_types=[
        pltpu.VMEM((EPW,), jnp.int32),
        pltpu.VMEM((EPW,), jnp.float32),
        pltpu.VMEM((NCHUNK, CH), jnp.int32),
    ] + [pltpu.VMEM((CH, D), jnp.bfloat16)] * NBUF
      + [pltpu.VMEM((CH, D), jnp.float32)] * NBUF + [
        pltpu.VMEM_SHARED((N_NODES, D), jnp.float32),
    ] + [pltpu.SemaphoreType.DMA] * (2 * NBUF),
    compiler_params=pltpu.CompilerParams(
        use_tc_tiling_on_sc=False, needs_layout_passes=False),
)


def _ileave_bf16(y):
    n = y.shape[0]
    return jnp.transpose(
        y.reshape(n, D // 32, 2, 16), (0, 1, 3, 2)).reshape(n, D).astype(
            jnp.bfloat16)


def _proj_body(x_ref, w_ref, o_ref, ob_ref):
    y = jnp.dot(x_ref[...], w_ref[...], preferred_element_type=jnp.float32)
    o_ref[...] = y
    ob_ref[...] = _ileave_bf16(y)


def _stage1_body(y_ref, p_ref, b1_ref, w2_ref, b2_ref, o_ref, ob_ref):
    u = jnp.maximum(y_ref[...] + p_ref[0] + p_ref[1] + b1_ref[...], 0.0)
    h = jnp.dot(u, w2_ref[...], preferred_element_type=jnp.float32) + b2_ref[...]
    h = jnp.maximum(h, 0.0)
    o_ref[...] = h
    ob_ref[...] = _ileave_bf16(h)


def _stage2_body(h_ref, p_ref, w1_ref, b1_ref, w2_ref, b2_ref, o_ref, ob_ref):
    z = h_ref[...] + p_ref[0] + p_ref[1]
    u = jnp.maximum(
        jnp.dot(z, w1_ref[...], preferred_element_type=jnp.float32) + b1_ref[...],
        0.0)
    h = jnp.dot(u, w2_ref[...], preferred_element_type=jnp.float32) + b2_ref[...]
    h = jnp.maximum(h, 0.0)
    o_ref[...] = h
    ob_ref[...] = _ileave_bf16(h)


def _stage3_body(h_ref, p_ref, w1_ref, b1_ref, w2_ref, b2_ref, o_ref):
    z = h_ref[...] + p_ref[0] + p_ref[1]
    u = jnp.maximum(
        jnp.dot(z, w1_ref[...], preferred_element_type=jnp.float32) + b1_ref[...],
        0.0)
    o_ref[...] = jnp.dot(
        u, w2_ref[...], preferred_element_type=jnp.float32) + b2_ref[...]


_dual_out = (jax.ShapeDtypeStruct((N_NODES, D), jnp.float32),
             jax.ShapeDtypeStruct((N_NODES, D), jnp.bfloat16))

BR = 1000


def _rb(cols):
    return pl.BlockSpec((BR, cols), lambda i: (i, 0))


def _pb():
    return pl.BlockSpec((NC, BR, D), lambda i: (0, i, 0))


def _full(shape):
    return pl.BlockSpec(shape, lambda i: tuple(0 for _ in shape))


def kernel(x, edge_index, edge_weight, W11, b11, W12, b12,
           W21, b21, W22, b22, W31, b31, W32, b32):
    src = edge_index[0].astype(jnp.int32)
    dst = edge_index[1].astype(jnp.int32).reshape(NW, NCHUNK, CH)
    w = edge_weight.astype(jnp.float32)
    grid = (N_NODES // BR,)

    y1, y1b = pl.pallas_call(
        _proj_body, grid=grid,
        in_specs=[_rb(128), _full((128, 64))],
        out_specs=[_rb(64), _rb(64)],
        out_shape=_dual_out)(x, W11)
    p1 = _seg_sum(y1b, src, dst, w)
    h1, h1b = pl.pallas_call(
        _stage1_body, grid=grid,
        in_specs=[_rb(64), _pb(), _full((1, 64)), _full((64, 64)),
                  _full((1, 64))],
        out_specs=[_rb(64), _rb(64)],
        out_shape=_dual_out)(
            y1, p1, b11.reshape(1, 64), W12, b12.reshape(1, 64))

    p2 = _seg_sum(h1b, src, dst, w)
    h2, h2b = pl.pallas_call(
        _stage2_body, grid=grid,
        in_specs=[_rb(64), _pb(), _full((64, 128)), _full((1, 128)),
                  _full((128, 64)), _full((1, 64))],
        out_specs=[_rb(64), _rb(64)],
        out_shape=_dual_out)(
            h1, p2, W21, b21.reshape(1, 128), W22, b22.reshape(1, 64))

    p3 = _seg_sum(h2b, src, dst, w)
    out = pl.pallas_call(
        _stage3_body, grid=grid,
        in_specs=[_rb(64), _pb(), _full((64, 128)), _full((1, 128)),
                  _full((128, 128)), _full((1, 128))],
        out_specs=_rb(128),
        out_shape=jax.ShapeDtypeStruct((N_NODES, 128), jnp.float32))(
            h2, p3, W31, b31.reshape(1, 128), W32, b32.reshape(1, 128))
    return out


# speedup vs baseline: 1.5913x; 1.5913x over previous
import jax
import jax.numpy as jnp
from jax import lax
from jax.experimental import pallas as pl
from jax.experimental.pallas import tpu as pltpu
from jax.experimental.pallas import tpu_sc as plsc

N_NODES = 10000
N_EDGES = 320000
D = 64
NC = 2
NS = 16
NW = NC * NS
EPW = N_EDGES // NW
CH = 80
NCHUNK = EPW // CH
RPT = N_NODES // NS
NBUF = 4
UNROLL = 8

_mesh = plsc.VectorSubcoreMesh(
    core_axis_name="c", subcore_axis_name="s", num_cores=NC, num_subcores=NS)


def _seg_body(y_hbm, src_hbm, dst_hbm, w_hbm, out_hbm,
              src_v, w_v, dsti_v, *rest):
    rows = list(rest[:NBUF])
    acc = rest[NBUF]
    gsem = list(rest[NBUF + 1:2 * NBUF + 1])
    ssem = list(rest[2 * NBUF + 1:3 * NBUF + 1])
    del rest
    c = lax.axis_index("c")
    s = lax.axis_index("s")
    wid = c * NS + s
    base = wid * EPW

    zvec = jnp.zeros((16,), jnp.float32)
    zbuf = rows[0]

    @pl.loop(0, CH)
    def _(r):
        for dd in range(D // 16):
            zbuf[r, pl.ds(dd * 16, 16)] = zvec

    @pl.loop(0, RPT // CH)
    def _(k):
        pltpu.sync_copy(zbuf, acc.at[pl.ds(s * RPT + k * CH, CH)])

    pltpu.sync_copy(zbuf.at[pl.ds(0, RPT % CH)],
                    acc.at[pl.ds(s * RPT + (RPT // CH) * CH, RPT % CH)])

    pltpu.sync_copy(src_hbm.at[pl.ds(base, EPW)], src_v)
    pltpu.sync_copy(w_hbm.at[pl.ds(base, EPW)], w_v)
    pltpu.sync_copy(dst_hbm.at[wid], dsti_v)

    plsc.subcore_barrier()

    def start_gather(k, b):
        pltpu.async_copy(y_hbm.at[src_v.at[pl.ds(k * CH, CH)]], rows[b],
                         gsem[b])

    def wait_gather(k, b):
        pltpu.make_async_copy(y_hbm.at[src_v.at[pl.ds(k * CH, CH)]], rows[b],
                              gsem[b]).wait()

    def start_scatter(k, b):
        pltpu.async_copy(rows[b], acc.at[dsti_v.at[k]], ssem[b], add=True)

    def wait_scatter(k, b):
        pltpu.make_async_copy(rows[b], acc.at[dsti_v.at[k]], ssem[b]).wait()

    def multiply(k, b):
        buf = rows[b]

        @plsc.parallel_loop(0, CH, unroll=UNROLL)
        def _(r):
            wb = plsc.load_gather(
                w_v, [jnp.full((16,), k * CH + r, dtype=jnp.int32)])
            for dd in range(D // 16):
                sl = pl.ds(dd * 16, 16)
                buf[r, sl] = buf[r, sl] * wb

    for b in range(NBUF - 1):
        start_gather(b, b)

    ngrp = (NCHUNK + NBUF - 1) // NBUF

    @pl.loop(0, ngrp)
    def _(j):
        for i in range(NBUF):
            k = j * NBUF + i

            @pl.when(k < NCHUNK)
            def _():
                wait_gather(k, i)
                multiply(k, i)
                start_scatter(k, i)

            @pl.when(jnp.logical_and(k >= 1, k <= NCHUNK - 1))
            def _():
                wait_scatter(k - 1, (i - 1) % NBUF)

            @pl.when(k + NBUF - 1 < NCHUNK)
            def _():
                start_gather(k + NBUF - 1, (i + NBUF - 1) % NBUF)

    wait_scatter(NCHUNK - 1, (NCHUNK - 1) % NBUF)

    plsc.subcore_barrier()
    pltpu.sync_copy(acc.at[pl.ds(s * RPT, RPT)],
                    out_hbm.at[c, pl.ds(s * RPT, RPT)])


_seg_sum = pl.kernel(
    _seg_body,
    out_type=jax.ShapeDtypeStruct((NC, N_NODES, D), jnp.float32),
    mesh=_mesh,
    scratch_types=[
        pltpu.VMEM((EPW,), jnp.int32),
        pltpu.VMEM((EPW,), jnp.float32),
        pltpu.VMEM((NCHUNK, CH), jnp.int32),
    ] + [pltpu.VMEM((CH, D), jnp.float32)] * NBUF + [
        pltpu.VMEM_SHARED((N_NODES, D), jnp.float32),
    ] + [pltpu.SemaphoreType.DMA] * (2 * NBUF),
    compiler_params=pltpu.CompilerParams(
        use_tc_tiling_on_sc=False, needs_layout_passes=False),
)


def _proj_body(x_ref, w_ref, o_ref):
    o_ref[...] = jnp.dot(x_ref[...], w_ref[...],
                         preferred_element_type=jnp.float32)


def _stage1_body(y_ref, p_ref, b1_ref, w2_ref, b2_ref, o_ref):
    u = jnp.maximum(y_ref[...] + p_ref[0] + p_ref[1] + b1_ref[...], 0.0)
    h = jnp.dot(u, w2_ref[...], preferred_element_type=jnp.float32) + b2_ref[...]
    o_ref[...] = jnp.maximum(h, 0.0)


def _stage2_body(h_ref, p_ref, w1_ref, b1_ref, w2_ref, b2_ref, o_ref):
    z = h_ref[...] + p_ref[0] + p_ref[1]
    u = jnp.maximum(
        jnp.dot(z, w1_ref[...], preferred_element_type=jnp.float32) + b1_ref[...],
        0.0)
    h = jnp.dot(u, w2_ref[...], preferred_element_type=jnp.float32) + b2_ref[...]
    o_ref[...] = jnp.maximum(h, 0.0)


def _stage3_body(h_ref, p_ref, w1_ref, b1_ref, w2_ref, b2_ref, o_ref):
    z = h_ref[...] + p_ref[0] + p_ref[1]
    u = jnp.maximum(
        jnp.dot(z, w1_ref[...], preferred_element_type=jnp.float32) + b1_ref[...],
        0.0)
    o_ref[...] = jnp.dot(
        u, w2_ref[...], preferred_element_type=jnp.float32) + b2_ref[...]


BR = 1000


def _rb(cols):
    return pl.BlockSpec((BR, cols), lambda i: (i, 0))


def _pb():
    return pl.BlockSpec((NC, BR, D), lambda i: (0, i, 0))


def _full(shape):
    return pl.BlockSpec(shape, lambda i: tuple(0 for _ in shape))


def kernel(x, edge_index, edge_weight, W11, b11, W12, b12,
           W21, b21, W22, b22, W31, b31, W32, b32):
    src = edge_index[0].astype(jnp.int32)
    dst = edge_index[1].astype(jnp.int32).reshape(NW, NCHUNK, CH)
    w = edge_weight.astype(jnp.float32)
    grid = (N_NODES // BR,)

    y1 = pl.pallas_call(
        _proj_body, grid=grid,
        in_specs=[_rb(128), _full((128, 64))],
        out_specs=_rb(64),
        out_shape=jax.ShapeDtypeStruct((N_NODES, D), jnp.float32))(x, W11)
    p1 = _seg_sum(y1, src, dst, w)
    h1 = pl.pallas_call(
        _stage1_body, grid=grid,
        in_specs=[_rb(64), _pb(), _full((1, 64)), _full((64, 64)),
                  _full((1, 64))],
        out_specs=_rb(64),
        out_shape=jax.ShapeDtypeStruct((N_NODES, D), jnp.float32))(
            y1, p1, b11.reshape(1, 64), W12, b12.reshape(1, 64))

    p2 = _seg_sum(h1, src, dst, w)
    h2 = pl.pallas_call(
        _stage2_body, grid=grid,
        in_specs=[_rb(64), _pb(), _full((64, 128)), _full((1, 128)),
                  _full((128, 64)), _full((1, 64))],
        out_specs=_rb(64),
        out_shape=jax.ShapeDtypeStruct((N_NODES, D), jnp.float32))(
            h1, p2, W21, b21.reshape(1, 128), W22, b22.reshape(1, 64))

    p3 = _seg_sum(h2, src, dst, w)
    out = pl.pallas_call(
        _stage3_body, grid=grid,
        in_specs=[_rb(64), _pb(), _full((64, 128)), _full((1, 128)),
                  _full((128, 128)), _full((1, 128))],
        out_specs=_rb(128),
        out_shape=jax.ShapeDtypeStruct((N_NODES, 128), jnp.float32))(
            h2, p3, W31, b31.reshape(1, 128), W32, b32.reshape(1, 128))
    return out


# speedup vs baseline: 1.6622x; 1.0445x over previous
import jax
import jax.numpy as jnp
from jax import lax
from jax.experimental import pallas as pl
from jax.experimental.pallas import tpu as pltpu
from jax.experimental.pallas import tpu_sc as plsc

N_NODES = 10000
N_EDGES = 320000
D = 64
NC = 2
NS = 16
NW = NC * NS
EPW = N_EDGES // NW
CH = 80
NCHUNK = EPW // CH
RPT = N_NODES // NS
NBUF = 4
UNROLL = 8

_mesh = plsc.VectorSubcoreMesh(
    core_axis_name="c", subcore_axis_name="s", num_cores=NC, num_subcores=NS)


def _seg_body(y_hbm, src_hbm, dst_hbm, w_hbm, out_hbm,
              src_v, w_v, dsti_v, *rest):
    rows = list(rest[:NBUF])
    acc = rest[NBUF]
    gsem = list(rest[NBUF + 1:2 * NBUF + 1])
    ssem = list(rest[2 * NBUF + 1:3 * NBUF + 1])
    del rest
    c = lax.axis_index("c")
    s = lax.axis_index("s")
    wid = c * NS + s
    base = wid * EPW

    zvec = jnp.zeros((16,), jnp.float32)
    zbuf = rows[0]

    @pl.loop(0, CH)
    def _(r):
        for dd in range(D // 16):
            zbuf[r, pl.ds(dd * 16, 16)] = zvec

    @pl.loop(0, RPT // CH)
    def _(k):
        pltpu.sync_copy(zbuf, acc.at[pl.ds(s * RPT + k * CH, CH)])

    pltpu.sync_copy(zbuf.at[pl.ds(0, RPT % CH)],
                    acc.at[pl.ds(s * RPT + (RPT // CH) * CH, RPT % CH)])

    pltpu.sync_copy(src_hbm.at[pl.ds(base, EPW)], src_v)
    pltpu.sync_copy(w_hbm.at[pl.ds(base, EPW)], w_v)
    pltpu.sync_copy(dst_hbm.at[wid], dsti_v)

    plsc.subcore_barrier()

    def start_gather(k, b):
        pltpu.async_copy(y_hbm.at[src_v.at[pl.ds(k * CH, CH)]], rows[b],
                         gsem[b])

    def wait_gather(k, b):
        pltpu.make_async_copy(y_hbm.at[src_v.at[pl.ds(k * CH, CH)]], rows[b],
                              gsem[b]).wait()

    def start_scatter(k, b):
        pltpu.async_copy(rows[b], acc.at[dsti_v.at[k]], ssem[b], add=True)

    def wait_scatter(k, b):
        pltpu.make_async_copy(rows[b], acc.at[dsti_v.at[k]], ssem[b]).wait()

    def multiply(k, b):
        buf = rows[b]

        @plsc.parallel_loop(0, CH, unroll=UNROLL)
        def _(r):
            wb = plsc.load_gather(
                w_v, [jnp.full((16,), k * CH + r, dtype=jnp.int32)])
            for dd in range(D // 16):
                sl = pl.ds(dd * 16, 16)
                buf[r, sl] = buf[r, sl] * wb

    for b in range(NBUF - 1):
        start_gather(b, b)

    ngrp = (NCHUNK + NBUF - 1) // NBUF

    @pl.loop(0, ngrp)
    def _(j):
        for i in range(NBUF):
            k = j * NBUF + i

            @pl.when(k < NCHUNK)
            def _():
                wait_gather(k, i)
                multiply(k, i)
                start_scatter(k, i)

            @pl.when(jnp.logical_and(k >= 1, k <= NCHUNK - 1))
            def _():
                wait_scatter(k - 1, (i - 1) % NBUF)

            @pl.when(k + NBUF - 1 < NCHUNK)
            def _():
                start_gather(k + NBUF - 1, (i + NBUF - 1) % NBUF)

    wait_scatter(NCHUNK - 1, (NCHUNK - 1) % NBUF)

    plsc.subcore_barrier()
    pltpu.sync_copy(acc.at[pl.ds(s * RPT, RPT)],
                    out_hbm.at[c, pl.ds(s * RPT, RPT)])


_seg_sum = pl.kernel(
    _seg_body,
    out_type=jax.ShapeDtypeStruct((NC, N_NODES, D), jnp.float32),
    mesh=_mesh,
    scratch_types=[
        pltpu.VMEM((EPW,), jnp.int32),
        pltpu.VMEM((EPW,), jnp.float32),
        pltpu.VMEM((NCHUNK, CH), jnp.int32),
    ] + [pltpu.VMEM((CH, D), jnp.float32)] * NBUF + [
        pltpu.VMEM_SHARED((N_NODES, D), jnp.float32),
    ] + [pltpu.SemaphoreType.DMA] * (2 * NBUF),
    compiler_params=pltpu.CompilerParams(
        use_tc_tiling_on_sc=False, needs_layout_passes=False),
)


def _proj_body(x_ref, w_ref, o_ref):
    o_ref[...] = jnp.dot(x_ref[...], w_ref[...],
                         preferred_element_type=jnp.float32)


def _stage1_body(y_ref, p_ref, b1_ref, w2_ref, b2_ref, o_ref):
    u = jnp.maximum(y_ref[...] + p_ref[0] + p_ref[1] + b1_ref[...], 0.0)
    h = jnp.dot(u, w2_ref[...], preferred_element_type=jnp.float32) + b2_ref[...]
    o_ref[...] = jnp.maximum(h, 0.0)


def _stage2_body(h_ref, p_ref, w1_ref, b1_ref, w2_ref, b2_ref, o_ref):
    z = h_ref[...] + p_ref[0] + p_ref[1]
    u = jnp.maximum(
        jnp.dot(z, w1_ref[...], preferred_element_type=jnp.float32) + b1_ref[...],
        0.0)
    h = jnp.dot(u, w2_ref[...], preferred_element_type=jnp.float32) + b2_ref[...]
    o_ref[...] = jnp.maximum(h, 0.0)


def _stage3_body(h_ref, p_ref, w1_ref, b1_ref, w2_ref, b2_ref, o_ref):
    z = h_ref[...] + p_ref[0] + p_ref[1]
    u = jnp.maximum(
        jnp.dot(z, w1_ref[...], preferred_element_type=jnp.float32) + b1_ref[...],
        0.0)
    o_ref[...] = jnp.dot(
        u, w2_ref[...], preferred_element_type=jnp.float32) + b2_ref[...]


def kernel(x, edge_index, edge_weight, W11, b11, W12, b12,
           W21, b21, W22, b22, W31, b31, W32, b32):
    src = edge_index[0].astype(jnp.int32)
    dst = edge_index[1].astype(jnp.int32).reshape(NW, NCHUNK, CH)
    w = edge_weight.astype(jnp.float32)

    y1 = pl.pallas_call(
        _proj_body,
        out_shape=jax.ShapeDtypeStruct((N_NODES, D), jnp.float32))(x, W11)
    p1 = _seg_sum(y1, src, dst, w)
    h1 = pl.pallas_call(
        _stage1_body,
        out_shape=jax.ShapeDtypeStruct((N_NODES, D), jnp.float32))(
            y1, p1, b11.reshape(1, 64), W12, b12.reshape(1, 64))

    p2 = _seg_sum(h1, src, dst, w)
    h2 = pl.pallas_call(
        _stage2_body,
        out_shape=jax.ShapeDtypeStruct((N_NODES, D), jnp.float32))(
            h1, p2, W21, b21.reshape(1, 128), W22, b22.reshape(1, 64))

    p3 = _seg_sum(h2, src, dst, w)
    out = pl.pallas_call(
        _stage3_body,
        out_shape=jax.ShapeDtypeStruct((N_NODES, 128), jnp.float32))(
            h2, p3, W31, b31.reshape(1, 128), W32, b32.reshape(1, 128))
    return out
